# baseline (speedup 1.0000x reference)
import jax
import jax.numpy as jnp
from jax import lax
from jax.experimental import pallas as pl
from jax.experimental.pallas import tpu as pltpu
from jax.experimental.pallas import tpu_sc as plsc

_LEVELS = 256
_SCALE = (_LEVELS - 1) / 2.0
_BIAS = _SCALE + 0.5

_B, _R, _C = 2, 8192, 2048
_NW = 32
_TPB = _NW // _B
_ROWS_PW = _R // _TPB
_CROWS = 8
_NCHUNK = _ROWS_PW // _CROWS
_UNROLL = 8
_HROWS = _CROWS // 2


def _body(x_hbm, lut_hbm, out_hbm, lut_v,
          in_v0, in_v1, out_v0, out_v1,
          isem0, isem1, osem0, osem1):
    wid = lax.axis_index("s") * 2 + lax.axis_index("c")
    d0 = wid // _TPB
    row0 = (wid % _TPB) * _ROWS_PW
    pltpu.sync_copy(lut_hbm, lut_v)

    in_bufs = (in_v0, in_v1)
    out_bufs = (out_v0, out_v1)
    isems = (isem0, isem1)
    osems = (osem0, osem1)

    def row(ci):
        return pl.multiple_of(row0 + ci * _CROWS, _CROWS)

    def start_in(ci, b):
        pltpu.async_copy(x_hbm.at[d0, pl.ds(row(ci), _CROWS), :],
                         in_bufs[b], isems[b])

    def wait_in(b):
        pltpu.make_async_copy(x_hbm.at[0, pl.ds(0, _CROWS), :],
                              in_bufs[b], isems[b]).wait()

    def start_out_half(ci, b, h):
        pltpu.async_copy(
            out_bufs[b].at[pl.ds(h * _HROWS, _HROWS), :],
            out_hbm.at[d0, pl.ds(row(ci) + h * _HROWS, _HROWS), :], osems[b])

    def wait_out(b):
        pltpu.make_async_copy(out_bufs[b],
                              out_hbm.at[0, pl.ds(0, _CROWS), :], osems[b]).wait()

    def compute_half(b, h):
        inb, outb = in_bufs[b], out_bufs[b]
        for r in range(h * _HROWS, (h + 1) * _HROWS):
            @plsc.parallel_loop(0, _C, step=16, unroll=_UNROLL)
            def _vec(i):
                v = inb[r, pl.ds(i, 16)]
                v = jnp.minimum(jnp.maximum(v, -1.0), 1.0)
                idx = (v * _SCALE + _BIAS).astype(jnp.int32)
                outb[r, pl.ds(i, 16)] = plsc.load_gather(lut_v, [idx])

    start_in(0, 0)
    start_in(1, 1)

    @pl.loop(0, _NCHUNK, step=2)
    def _main(ci):
        for b in range(2):
            cur = ci + b
            wait_in(b)

            @pl.when(cur >= 2)
            def _():
                wait_out(b)

            compute_half(b, 0)
            start_out_half(cur, b, 0)
            compute_half(b, 1)
            start_out_half(cur, b, 1)

            @pl.when(cur + 2 < _NCHUNK)
            def _():
                start_in(cur + 2, b)

    wait_out(0)
    wait_out(1)


@jax.jit
def kernel(x, lut):
    mesh = plsc.VectorSubcoreMesh(core_axis_name="c", subcore_axis_name="s")
    f = pl.kernel(
        _body,
        out_type=jax.ShapeDtypeStruct((_B, _R, _C), jnp.float32),
        mesh=mesh,
        scratch_types=(
            [pltpu.VMEM((_LEVELS,), jnp.float32)]
            + [pltpu.VMEM((_CROWS, _C), jnp.float32) for _ in range(4)]
            + [pltpu.SemaphoreType.DMA for _ in range(4)]
        ),
        compiler_params=pltpu.CompilerParams(needs_layout_passes=False),
    )
    return f(x, lut)


# speedup vs baseline: 1.0631x; 1.0631x over previous
import jax
import jax.numpy as jnp
from jax import lax
from jax.experimental import pallas as pl
from jax.experimental.pallas import tpu as pltpu
from jax.experimental.pallas import tpu_sc as plsc

_LEVELS = 256
_SCALE = (_LEVELS - 1) / 2.0
_BIAS = _SCALE + 0.5

_B, _R, _C = 2, 8192, 2048
_NW = 32
_TPB = _NW // _B
_ROWS_PW = _R // _TPB
_CROWS = 8
_NCHUNK = _ROWS_PW // _CROWS
_UNROLL = 8


def _body(x_hbm, lut_hbm, out_hbm, lut_v,
          in_v0, in_v1, out_v0, out_v1,
          isem0, isem1, osem0, osem1):
    wid = lax.axis_index("s") * 2 + lax.axis_index("c")
    d0 = wid // _TPB
    row0 = (wid % _TPB) * _ROWS_PW
    pltpu.sync_copy(lut_hbm, lut_v)

    in_bufs = (in_v0, in_v1)
    out_bufs = (out_v0, out_v1)
    isems = (isem0, isem1)
    osems = (osem0, osem1)

    def row(ci):
        return pl.multiple_of(row0 + ci * _CROWS, _CROWS)

    def start_in(ci, b):
        pltpu.async_copy(x_hbm.at[d0, pl.ds(row(ci), _CROWS), :],
                         in_bufs[b], isems[b])

    def wait_in(b):
        pltpu.make_async_copy(x_hbm.at[0, pl.ds(0, _CROWS), :],
                              in_bufs[b], isems[b]).wait()

    def start_out(ci, b):
        pltpu.async_copy(out_bufs[b],
                         out_hbm.at[d0, pl.ds(row(ci), _CROWS), :], osems[b])

    def wait_out(b):
        pltpu.make_async_copy(out_bufs[b],
                              out_hbm.at[0, pl.ds(0, _CROWS), :], osems[b]).wait()

    def compute(b):
        inb, outb = in_bufs[b], out_bufs[b]
        for r in range(_CROWS):
            @plsc.parallel_loop(0, _C, step=16, unroll=_UNROLL)
            def _vec(i):
                v = inb[r, pl.ds(i, 16)]
                v = jnp.minimum(jnp.maximum(v, -1.0), 1.0)
                idx = (v * _SCALE + _BIAS).astype(jnp.int32)
                outb[r, pl.ds(i, 16)] = plsc.load_gather(lut_v, [idx])

    start_in(0, 0)
    start_in(1, 1)

    @pl.loop(0, _NCHUNK, step=2)
    def _main(ci):
        for b in range(2):
            cur = ci + b
            wait_in(b)

            @pl.when(cur >= 2)
            def _():
                wait_out(b)

            compute(b)
            start_out(cur, b)

            @pl.when(cur + 2 < _NCHUNK)
            def _():
                start_in(cur + 2, b)

    wait_out(0)
    wait_out(1)


@jax.jit
def kernel(x, lut):
    mesh = plsc.VectorSubcoreMesh(core_axis_name="c", subcore_axis_name="s")
    f = pl.kernel(
        _body,
        out_type=jax.ShapeDtypeStruct((_B, _R, _C), jnp.float32),
        mesh=mesh,
        scratch_types=(
            [pltpu.VMEM((_LEVELS,), jnp.float32)]
            + [pltpu.VMEM((_CROWS, _C), jnp.float32) for _ in range(4)]
            + [pltpu.SemaphoreType.DMA for _ in range(4)]
        ),
        compiler_params=pltpu.CompilerParams(needs_layout_passes=False),
    )
    return f(x, lut)
